# baseline (speedup 1.0000x reference)
import dataclasses

import jax
import jax.numpy as jnp
from jax import lax
from jax.experimental import pallas as pl
from jax.experimental.pallas import tpu as pltpu
from jax.experimental.pallas import tpu_sc as plsc

def _sc_compiler_params():
    cp = pltpu.CompilerParams(use_tc_tiling_on_sc=False)
    if "needs_layout_passes" in pltpu.CompilerParams.__dataclass_fields__:
        cp = dataclasses.replace(cp, needs_layout_passes=False)
    return cp


_V = 100000
_E = 32
_L = 16


def kernel(idx, directions):
    b, s = idx.shape
    assert directions.shape == (_V, _E)
    assert b % 4096 == 0
    nq = b // 4096

    table_flat = directions.T.reshape(_V * _E)
    idx_flat = idx.T.reshape(b * s)

    mesh = plsc.VectorSubcoreMesh(core_axis_name="core", subcore_axis_name="subcore")

    @pl.kernel(out_type=jax.ShapeDtypeStruct((s, _E // 8, b // 128, 1024),
                                             directions.dtype),
               mesh=mesh,
               scratch_types=[
                   pltpu.VMEM((_V,), jnp.float32),
               ],
               compiler_params=_sc_compiler_params())
    def gather_kernel(table_hbm, idx_hbm, out_hbm, row_v):
        e = lax.axis_index("core") * 16 + lax.axis_index("subcore")
        e_hi = e // 8
        e_lo = e % 8
        pltpu.sync_copy(table_hbm.at[pl.ds(e * _V, _V)], row_v)

        def body(i_vmem, o_vmem):
            @pl.loop(0, 32)
            def _(r):
                @pl.loop(0, 128, step=_L)
                def _(c):
                    ids = i_vmem[pl.ds(r * 128 + c, _L)]
                    o_vmem[0, 0, r, pl.ds(c, _L)] = plsc.load_gather(row_v, [ids])

        pltpu.emit_pipeline(
            body,
            grid=(s * nq,),
            in_specs=[pl.BlockSpec((4096,), index_map=lambda t: (t,))],
            out_specs=[pl.BlockSpec((1, 1, 32, 128),
                                    index_map=lambda t: (t // nq, e_hi, t % nq, e_lo))],
        )(idx_hbm, out_hbm)

    out5 = gather_kernel(table_flat, idx_flat)
    return (out5.reshape(s, _E // 8, b // 128, 8, 128)
            .transpose(2, 4, 0, 1, 3)
            .reshape(b, s, _E))


# speedup vs baseline: 2.4630x; 2.4630x over previous
import dataclasses

import jax
import jax.numpy as jnp
from jax import lax
from jax.experimental import pallas as pl
from jax.experimental.pallas import tpu as pltpu
from jax.experimental.pallas import tpu_sc as plsc

def _sc_compiler_params():
    cp = pltpu.CompilerParams(use_tc_tiling_on_sc=False)
    if "needs_layout_passes" in pltpu.CompilerParams.__dataclass_fields__:
        cp = dataclasses.replace(cp, needs_layout_passes=False)
    return cp


_V = 100000
_E = 32
_L = 16


def kernel(idx, directions):
    b, s = idx.shape
    assert directions.shape == (_V, _E)
    assert b % 4096 == 0
    nq = b // 4096

    table_flat = directions.T.reshape(_V * _E)
    idx_flat = idx.T.reshape(b * s)

    mesh = plsc.VectorSubcoreMesh(core_axis_name="core", subcore_axis_name="subcore")

    @pl.kernel(out_type=jax.ShapeDtypeStruct((s, _E // 8, b // 128, 1024),
                                             directions.dtype),
               mesh=mesh,
               scratch_types=[
                   pltpu.VMEM((_V,), jnp.float32),
               ],
               compiler_params=_sc_compiler_params())
    def gather_kernel(table_hbm, idx_hbm, out_hbm, row_v):
        e = lax.axis_index("core") * 16 + lax.axis_index("subcore")
        e_hi = e // 8
        e_lo = e % 8
        pltpu.sync_copy(table_hbm.at[pl.ds(e * _V, _V)], row_v)

        def body(i_vmem, o_vmem):
            @pl.loop(0, 32)
            def _(r):
                @plsc.parallel_loop(0, 128, step=_L, unroll=8)
                def _(c):
                    ids = i_vmem[pl.ds(r * 128 + c, _L)]
                    o_vmem[0, 0, r, pl.ds(c, _L)] = plsc.load_gather(row_v, [ids])

        pltpu.emit_pipeline(
            body,
            grid=(s * nq,),
            in_specs=[pl.BlockSpec((4096,), index_map=lambda t: (t,))],
            out_specs=[pl.BlockSpec((1, 1, 32, 128),
                                    index_map=lambda t: (t // nq, e_hi, t % nq, e_lo))],
        )(idx_hbm, out_hbm)

    out5 = gather_kernel(table_flat, idx_flat)
    return (out5.reshape(s, _E // 8, b // 128, 8, 128)
            .transpose(2, 4, 0, 1, 3)
            .reshape(b, s, _E))


# speedup vs baseline: 2.4684x; 1.0022x over previous
import dataclasses

import jax
import jax.numpy as jnp
from jax import lax
from jax.experimental import pallas as pl
from jax.experimental.pallas import tpu as pltpu
from jax.experimental.pallas import tpu_sc as plsc

def _sc_compiler_params():
    cp = pltpu.CompilerParams(use_tc_tiling_on_sc=False)
    if "needs_layout_passes" in pltpu.CompilerParams.__dataclass_fields__:
        cp = dataclasses.replace(cp, needs_layout_passes=False)
    return cp


_V = 100000
_E = 32
_L = 16


def kernel(idx, directions):
    b, s = idx.shape
    assert directions.shape == (_V, _E)
    assert b % 4096 == 0
    nq = b // 4096

    table_flat = directions.T.reshape(_V * _E)
    idx_flat = idx.T.reshape(b * s)

    mesh = plsc.VectorSubcoreMesh(core_axis_name="core", subcore_axis_name="subcore")

    @pl.kernel(out_type=jax.ShapeDtypeStruct((s, _E // 8, b // 128, 1024),
                                             directions.dtype),
               mesh=mesh,
               scratch_types=[
                   pltpu.VMEM((_V,), jnp.float32),
               ],
               compiler_params=_sc_compiler_params())
    def gather_kernel(table_hbm, idx_hbm, out_hbm, row_v):
        e = lax.axis_index("core") * 16 + lax.axis_index("subcore")
        e_hi = e // 8
        e_lo = e % 8
        pltpu.sync_copy(table_hbm.at[pl.ds(e * _V, _V)], row_v)

        def body(i_vmem, o_vmem):
            @plsc.parallel_loop(0, 4096, step=_L, unroll=16)
            def _(c):
                ids = i_vmem[pl.ds(c, _L)]
                o_vmem[0, 0, c // 128, pl.ds(c % 128, _L)] = plsc.load_gather(row_v, [ids])

        pltpu.emit_pipeline(
            body,
            grid=(s * nq,),
            in_specs=[pl.BlockSpec((4096,), index_map=lambda t: (t,))],
            out_specs=[pl.BlockSpec((1, 1, 32, 128),
                                    index_map=lambda t: (t // nq, e_hi, t % nq, e_lo))],
        )(idx_hbm, out_hbm)

    out5 = gather_kernel(table_flat, idx_flat)
    return (out5.reshape(s, _E // 8, b // 128, 8, 128)
            .transpose(2, 4, 0, 1, 3)
            .reshape(b, s, _E))
